# baseline (speedup 1.0000x reference)
import functools

import jax
import jax.numpy as jnp
from jax import lax
from jax.experimental import pallas as pl
from jax.experimental.pallas import tpu as pltpu
from jax.experimental.pallas import tpu_sc as plsc

N = 10000
DIM_IN = 11
DIM_H = 256
NUM_GRAPHS = 64

NPAD = 10240
R = 1024
GRID = NPAD // R
NTILE = 16
C = 128
NC = 84
T = NC * C
EP = 16 * T
RPT = NPAD // NTILE
HALF = 128


def _prep1_body(x_ref, w_ref, av_ref, h_ref, aa_ref):
    h = jnp.dot(x_ref[...], w_ref[...], preferred_element_type=jnp.float32,
                precision=lax.Precision.HIGHEST)
    h_ref[0] = h[:, :HALF].astype(jnp.bfloat16)
    h_ref[1] = h[:, HALF:].astype(jnp.bfloat16)
    aa_ref[0] = jnp.sum(h * av_ref[0][None, :], axis=-1)
    aa_ref[1] = jnp.sum(h * av_ref[1][None, :], axis=-1)


def _prep23_body(prev_ref, b_ref, w_ref, av_ref, h_ref, aa_ref):
    feat = jnp.concatenate([prev_ref[0], prev_ref[1]], axis=-1) + b_ref[...]
    feat = jnp.maximum(feat, 0.0)
    h = jnp.dot(feat, w_ref[...], preferred_element_type=jnp.float32,
                precision=lax.Precision.HIGHEST)
    h_ref[0] = h[:, :HALF].astype(jnp.bfloat16)
    h_ref[1] = h[:, HALF:].astype(jnp.bfloat16)
    aa_ref[0] = jnp.sum(h * av_ref[0][None, :], axis=-1)
    aa_ref[1] = jnp.sum(h * av_ref[1][None, :], axis=-1)


_prep_out = [
    jax.ShapeDtypeStruct((2, NPAD, HALF), jnp.bfloat16),
    jax.ShapeDtypeStruct((2, NPAD), jnp.float32),
]
_prep_out_specs = [
    pl.BlockSpec((2, R, HALF), lambda i: (0, i, 0)),
    pl.BlockSpec((2, R), lambda i: (0, i)),
]

_prep1 = pl.pallas_call(
    _prep1_body,
    grid=(GRID,),
    in_specs=[
        pl.BlockSpec((R, DIM_IN), lambda i: (i, 0)),
        pl.BlockSpec((DIM_IN, DIM_H), lambda i: (0, 0)),
        pl.BlockSpec((2, DIM_H), lambda i: (0, 0)),
    ],
    out_specs=_prep_out_specs,
    out_shape=_prep_out,
)

_prep23 = pl.pallas_call(
    _prep23_body,
    grid=(GRID,),
    in_specs=[
        pl.BlockSpec((2, R, HALF), lambda i: (0, i, 0)),
        pl.BlockSpec((1, DIM_H), lambda i: (0, 0)),
        pl.BlockSpec((DIM_H, DIM_H), lambda i: (0, 0)),
        pl.BlockSpec((2, DIM_H), lambda i: (0, 0)),
    ],
    out_specs=_prep_out_specs,
    out_shape=_prep_out,
)


def _final_body(prev_ref, b_ref, batch_ref, lw_ref, lb_ref, out_ref,
                pooled_ref, counts_ref):
    i = pl.program_id(0)

    @pl.when(i == 0)
    def _init():
        pooled_ref[...] = jnp.zeros_like(pooled_ref)
        counts_ref[...] = jnp.zeros_like(counts_ref)

    feat = jnp.concatenate([prev_ref[0], prev_ref[1]], axis=-1) + b_ref[...]
    gids = lax.broadcasted_iota(jnp.int32, (NUM_GRAPHS, R), 0)
    onehot = (gids == batch_ref[...]).astype(jnp.float32)
    pooled_ref[...] += jnp.dot(onehot, feat, preferred_element_type=jnp.float32,
                               precision=lax.Precision.HIGHEST)
    counts_ref[...] += jnp.sum(onehot, axis=1, keepdims=True)

    @pl.when(i == GRID - 1)
    def _fin():
        cnt = jnp.maximum(counts_ref[...], 1.0)
        pooled = pooled_ref[...] / cnt
        out_ref[...] = (jnp.sum(pooled * lw_ref[...], axis=-1, keepdims=True)
                        + lb_ref[...])


_final = pl.pallas_call(
    _final_body,
    grid=(GRID,),
    in_specs=[
        pl.BlockSpec((2, R, HALF), lambda i: (0, i, 0)),
        pl.BlockSpec((1, DIM_H), lambda i: (0, 0)),
        pl.BlockSpec((1, R), lambda i: (0, i)),
        pl.BlockSpec((1, DIM_H), lambda i: (0, 0)),
        pl.BlockSpec((1, 1), lambda i: (0, 0)),
    ],
    out_specs=pl.BlockSpec((NUM_GRAPHS, 1), lambda i: (0, 0)),
    out_shape=jax.ShapeDtypeStruct((NUM_GRAPHS, 1), jnp.float32),
    scratch_shapes=[
        pltpu.VMEM((NUM_GRAPHS, DIM_H), jnp.float32),
        pltpu.VMEM((NUM_GRAPHS, 1), jnp.float32),
    ],
)


NC2 = NC // 2
KCH = 12
SUB = 32


def _sc_w_body(s_hbm, d_hbm, as_hbm, ad_hbm, w_hbm, den2_hbm,
               den_all, as_v, ad_v, sidx_v, didx_v, w_v, den_t,
               den_sum, den_tmp):
    cid = lax.axis_index("c")
    sid = lax.axis_index("s")
    zeros16 = jnp.zeros((16,), jnp.float32)

    pltpu.sync_copy(as_hbm, as_v)
    pltpu.sync_copy(ad_hbm, ad_v)
    pltpu.sync_copy(s_hbm.at[sid, pl.ds(cid * NC2, NC2)], sidx_v)
    pltpu.sync_copy(d_hbm.at[sid, pl.ds(cid * NC2, NC2)], didx_v)

    def _zd(i, _):
        den_t[pl.ds(i * 16, 16)] = zeros16
        return 0
    lax.fori_loop(0, NPAD // 16, _zd, 0)

    def _wchunk(jc, _):
        def _wgrp(g, _):
            sl = pl.ds(g * 16, 16)
            sv = sidx_v[jc, sl]
            dv = didx_v[jc, sl]
            al = plsc.load_gather(as_v, [sv]) + plsc.load_gather(ad_v, [dv])
            al = jnp.where(al > 0.0, al, al * 0.2)
            w = jnp.exp(al)
            w_v[jc, sl] = w
            plsc.addupdate_scatter(den_t, [dv], w)
            return 0
        lax.fori_loop(0, C // 16, _wgrp, 0)
        return 0
    lax.fori_loop(0, NC2, _wchunk, 0)

    pltpu.sync_copy(w_v, w_hbm.at[sid, pl.ds(cid * NC2, NC2)])
    pltpu.sync_copy(den_t, den_all.at[sid])
    plsc.subcore_barrier()

    base = sid * RPT

    def _zs(i, _):
        den_sum[pl.ds(i * 16, 16)] = zeros16
        return 0
    lax.fori_loop(0, RPT // 16, _zs, 0)

    for t in range(NTILE):
        pltpu.sync_copy(den_all.at[t, pl.ds(base, RPT)], den_tmp)

        def _acc(i, _):
            sl = pl.ds(i * 16, 16)
            den_sum[sl] = den_sum[sl] + den_tmp[sl]
            return 0
        lax.fori_loop(0, RPT // 16, _acc, 0)

    pltpu.sync_copy(den_sum, den2_hbm.at[cid, pl.ds(base, RPT)])


_sc_w = pl.kernel(
    _sc_w_body,
    out_type=(
        jax.ShapeDtypeStruct((NTILE, NC, C), jnp.float32),
        jax.ShapeDtypeStruct((2, NPAD), jnp.float32),
    ),
    mesh=plsc.VectorSubcoreMesh(core_axis_name="c", subcore_axis_name="s"),
    compiler_params=pltpu.CompilerParams(needs_layout_passes=False, use_tc_tiling_on_sc=False),
    scratch_types=[
        pltpu.VMEM_SHARED((NTILE, NPAD), jnp.float32),
        pltpu.VMEM((NPAD,), jnp.float32),
        pltpu.VMEM((NPAD,), jnp.float32),
        pltpu.VMEM((NC2, C), jnp.int32),
        pltpu.VMEM((NC2, C), jnp.int32),
        pltpu.VMEM((NC2, C), jnp.float32),
        pltpu.VMEM((NPAD,), jnp.float32),
        pltpu.VMEM((RPT,), jnp.float32),
        pltpu.VMEM((RPT,), jnp.float32),
    ],
)


def _sc_agg_body(s_hbm, d4_hbm, w_hbm, h_hbm, den2_hbm, out_hbm,
                 num_s, sadj_v, didx4_v, w_v, gb0_v, gb1_v, fb_v, den_sum,
                 den_tmp, gsem0, gsem1, ssem0):
    cid = lax.axis_index("c")
    sid = lax.axis_index("s")
    zeros16 = jnp.zeros((16,), jnp.float32)
    npad_off = cid * NPAD
    gb = (gb0_v, gb1_v)
    gsem = (gsem0, gsem1)
    lane = lax.broadcasted_iota(jnp.int32, (16,), 0)

    def _start_gather(j, b):
        pltpu.make_async_copy(h_hbm.at[sadj_v.at[j]], gb[b], gsem[b]).start()

    def _wait_gather(j, b):
        pltpu.make_async_copy(h_hbm.at[sadj_v.at[j]], gb[b], gsem[b]).wait()

    def _scale_block(jj, b, q):
        gv = gb[b]

        def _e(e, _):
            row = q * SUB + e
            rfull = jnp.full((16,), row, jnp.int32)
            wb = plsc.load_gather(
                w_v, [jnp.full((16,), jj, jnp.int32),
                      jnp.full((16,), row, jnp.int32)])
            for k in range(HALF // 32):
                v = gv[row, pl.ds(k * 32, 32)]
                a, bb = plsc.unpack(v, format=plsc.PackFormat.INTERLEAVED,
                                    preferred_element_type=jnp.float32)
                plsc.store_scatter(fb_v, [rfull, k * 32 + 2 * lane], a * wb)
                plsc.store_scatter(fb_v, [rfull, k * 32 + 2 * lane + 1],
                                   bb * wb)
            return 0
        lax.fori_loop(0, SUB, _e, 0)

    def _chunk_body(j, b):
        descs = []
        for q in range(C // SUB):
            _scale_block(j, b, q)
            descs.append(pltpu.async_copy(
                fb_v.at[pl.ds(q * SUB, SUB)],
                num_s.at[didx4_v.at[(C // SUB) * j + q]],
                ssem0, add=True))
        for dsc in descs:
            dsc.wait()

    def _zr(r, _):
        for k in range(8):
            fb_v[r, pl.ds(k * 16, 16)] = zeros16
        return 0
    lax.fori_loop(0, C, _zr, 0)
    for j in range(RPT // C):
        pltpu.sync_copy(fb_v, num_s.at[pl.ds(sid * RPT + j * C, C)])
    plsc.subcore_barrier()

    def _group(g, _):
        pltpu.sync_copy(s_hbm.at[sid, pl.ds(g * KCH, KCH)], sadj_v)
        pltpu.sync_copy(d4_hbm.at[sid, pl.ds(g * KCH * (C // SUB),
                                             KCH * (C // SUB))], didx4_v)
        pltpu.sync_copy(w_hbm.at[sid, pl.ds(g * KCH, KCH)], w_v)

        def _adj(i, _):
            jj = i // (C // 16)
            sl = pl.ds((i % (C // 16)) * 16, 16)
            sadj_v[jj, sl] = sadj_v[jj, sl] + npad_off
            return 0
        lax.fori_loop(0, KCH * (C // 16), _adj, 0)

        _start_gather(0, 0)

        def _pair(p, _):
            _start_gather(2 * p + 1, 1)
            _wait_gather(2 * p, 0)
            _chunk_body(2 * p, 0)

            @pl.when(p < KCH // 2 - 1)
            def _():
                _start_gather(2 * p + 2, 0)
            _wait_gather(2 * p + 1, 1)
            _chunk_body(2 * p + 1, 1)
            return 0
        lax.fori_loop(0, KCH // 2, _pair, 0)
        return 0
    lax.fori_loop(0, NC // KCH, _group, 0)
    plsc.subcore_barrier()

    base = sid * RPT
    pltpu.sync_copy(den2_hbm.at[0, pl.ds(base, RPT)], den_sum)
    pltpu.sync_copy(den2_hbm.at[1, pl.ds(base, RPT)], den_tmp)

    def _accd(i, _):
        sl = pl.ds(i * 16, 16)
        den_sum[sl] = den_sum[sl] + den_tmp[sl]
        return 0
    lax.fori_loop(0, RPT // 16, _accd, 0)

    for j in range(RPT // C):
        pltpu.sync_copy(num_s.at[pl.ds(base + j * C, C)], fb_v)

        def _div(r, _):
            db = plsc.load_gather(
                den_sum, [jnp.full((16,), j * C + r, jnp.int32)]) + 1e-16
            for k in range(8):
                sl = pl.ds(k * 16, 16)
                fb_v[r, sl] = fb_v[r, sl] / db
            return 0
        lax.fori_loop(0, C, _div, 0)
        pltpu.sync_copy(fb_v,
                        out_hbm.at[pl.ds(npad_off + base + j * C, C)])


_sc_agg = pl.kernel(
    _sc_agg_body,
    out_type=jax.ShapeDtypeStruct((2 * NPAD, HALF), jnp.float32),
    mesh=plsc.VectorSubcoreMesh(core_axis_name="c", subcore_axis_name="s"),
    compiler_params=pltpu.CompilerParams(needs_layout_passes=False, use_tc_tiling_on_sc=False),
    scratch_types=[
        pltpu.VMEM_SHARED((NPAD, HALF), jnp.float32),
        pltpu.VMEM((KCH, C), jnp.int32),
        pltpu.VMEM((KCH * (C // SUB), SUB), jnp.int32),
        pltpu.VMEM((KCH, C), jnp.float32),
        pltpu.VMEM((C, HALF), jnp.bfloat16),
        pltpu.VMEM((C, HALF), jnp.bfloat16),
        pltpu.VMEM((C, HALF), jnp.float32),
        pltpu.VMEM((RPT,), jnp.float32),
        pltpu.VMEM((RPT,), jnp.float32),
        pltpu.SemaphoreType.DMA,
        pltpu.SemaphoreType.DMA,
        pltpu.SemaphoreType.DMA,
    ],
)


def _sc_edge(s, d, d4, h_flat, as_a, ad_a):
    w, den2 = _sc_w(s, d, as_a, ad_a)
    return _sc_agg(s, d4, w, h_flat, den2)


def kernel(x, edge_index, batch, W1, a_src1, a_dst1, b1, W2, a_src2, a_dst2,
           b2, W3, a_src3, a_dst3, b3, lin_W, lin_b):
    f32 = jnp.float32
    loop = jnp.arange(N, dtype=jnp.int32)
    s = jnp.concatenate([edge_index[0].astype(jnp.int32), loop])
    d = jnp.concatenate([edge_index[1].astype(jnp.int32), loop])
    pad = EP - s.shape[0]
    s = jnp.concatenate([s, jnp.full((pad,), N, jnp.int32)]).reshape(NTILE, NC, C)
    d_flat = jnp.concatenate([d, jnp.full((pad,), N, jnp.int32)])
    d = d_flat.reshape(NTILE, NC, C)
    d4 = d_flat.reshape(NTILE, NC * (C // SUB), SUB)

    x_pad = jnp.zeros((NPAD, DIM_IN), f32).at[:N].set(x)
    batch_p = jnp.full((1, NPAD), NUM_GRAPHS, jnp.int32)
    batch_p = batch_p.at[0, :N].set(batch.astype(jnp.int32))

    av1 = jnp.stack([a_src1, a_dst1])
    av2 = jnp.stack([a_src2, a_dst2])
    av3 = jnp.stack([a_src3, a_dst3])

    h, aa = _prep1(x_pad, W1, av1)
    o = _sc_edge(s, d, d4, h.reshape(2 * NPAD, HALF), aa[0], aa[1])
    h, aa = _prep23(o.reshape(2, NPAD, HALF), b1.reshape(1, DIM_H), W2, av2)
    o = _sc_edge(s, d, d4, h.reshape(2 * NPAD, HALF), aa[0], aa[1])
    h, aa = _prep23(o.reshape(2, NPAD, HALF), b2.reshape(1, DIM_H), W3, av3)
    o = _sc_edge(s, d, d4, h.reshape(2 * NPAD, HALF), aa[0], aa[1])
    return _final(o.reshape(2, NPAD, HALF), b3.reshape(1, DIM_H), batch_p,
                  lin_W.reshape(1, DIM_H), lin_b.reshape(1, 1))


# speedup vs baseline: 1.1713x; 1.1713x over previous
import functools

import jax
import jax.numpy as jnp
from jax import lax
from jax.experimental import pallas as pl
from jax.experimental.pallas import tpu as pltpu
from jax.experimental.pallas import tpu_sc as plsc

N = 10000
DIM_IN = 11
DIM_H = 256
NUM_GRAPHS = 64

NPAD = 10240
R = 1024
GRID = NPAD // R
NTILE = 16
C = 128
NC = 84
T = NC * C
EP = 16 * T
RPT = NPAD // NTILE
HALF = 128


def _prep1_body(x_ref, w_ref, av_ref, h_ref, aa_ref):
    h = jnp.dot(x_ref[...], w_ref[...], preferred_element_type=jnp.float32,
                precision=lax.Precision.HIGHEST)
    h_ref[0] = h[:, :HALF]
    h_ref[1] = h[:, HALF:]
    aa_ref[0] = jnp.sum(h * av_ref[0][None, :], axis=-1)
    aa_ref[1] = jnp.sum(h * av_ref[1][None, :], axis=-1)


def _prep23_body(prev_ref, b_ref, w_ref, av_ref, h_ref, aa_ref):
    feat = jnp.concatenate([prev_ref[0], prev_ref[1]], axis=-1) + b_ref[...]
    feat = jnp.maximum(feat, 0.0)
    h = jnp.dot(feat, w_ref[...], preferred_element_type=jnp.float32,
                precision=lax.Precision.HIGHEST)
    h_ref[0] = h[:, :HALF]
    h_ref[1] = h[:, HALF:]
    aa_ref[0] = jnp.sum(h * av_ref[0][None, :], axis=-1)
    aa_ref[1] = jnp.sum(h * av_ref[1][None, :], axis=-1)


_prep_out = [
    jax.ShapeDtypeStruct((2, NPAD, HALF), jnp.float32),
    jax.ShapeDtypeStruct((2, NPAD), jnp.float32),
]
_prep_out_specs = [
    pl.BlockSpec((2, R, HALF), lambda i: (0, i, 0)),
    pl.BlockSpec((2, R), lambda i: (0, i)),
]

_prep1 = pl.pallas_call(
    _prep1_body,
    grid=(GRID,),
    in_specs=[
        pl.BlockSpec((R, DIM_IN), lambda i: (i, 0)),
        pl.BlockSpec((DIM_IN, DIM_H), lambda i: (0, 0)),
        pl.BlockSpec((2, DIM_H), lambda i: (0, 0)),
    ],
    out_specs=_prep_out_specs,
    out_shape=_prep_out,
)

_prep23 = pl.pallas_call(
    _prep23_body,
    grid=(GRID,),
    in_specs=[
        pl.BlockSpec((2, R, HALF), lambda i: (0, i, 0)),
        pl.BlockSpec((1, DIM_H), lambda i: (0, 0)),
        pl.BlockSpec((DIM_H, DIM_H), lambda i: (0, 0)),
        pl.BlockSpec((2, DIM_H), lambda i: (0, 0)),
    ],
    out_specs=_prep_out_specs,
    out_shape=_prep_out,
)


def _final_body(prev_ref, b_ref, batch_ref, lw_ref, lb_ref, out_ref,
                pooled_ref, counts_ref):
    i = pl.program_id(0)

    @pl.when(i == 0)
    def _init():
        pooled_ref[...] = jnp.zeros_like(pooled_ref)
        counts_ref[...] = jnp.zeros_like(counts_ref)

    feat = jnp.concatenate([prev_ref[0], prev_ref[1]], axis=-1) + b_ref[...]
    gids = lax.broadcasted_iota(jnp.int32, (NUM_GRAPHS, R), 0)
    onehot = (gids == batch_ref[...]).astype(jnp.float32)
    pooled_ref[...] += jnp.dot(onehot, feat, preferred_element_type=jnp.float32,
                               precision=lax.Precision.HIGHEST)
    counts_ref[...] += jnp.sum(onehot, axis=1, keepdims=True)

    @pl.when(i == GRID - 1)
    def _fin():
        cnt = jnp.maximum(counts_ref[...], 1.0)
        pooled = pooled_ref[...] / cnt
        out_ref[...] = (jnp.sum(pooled * lw_ref[...], axis=-1, keepdims=True)
                        + lb_ref[...])


_final = pl.pallas_call(
    _final_body,
    grid=(GRID,),
    in_specs=[
        pl.BlockSpec((2, R, HALF), lambda i: (0, i, 0)),
        pl.BlockSpec((1, DIM_H), lambda i: (0, 0)),
        pl.BlockSpec((1, R), lambda i: (0, i)),
        pl.BlockSpec((1, DIM_H), lambda i: (0, 0)),
        pl.BlockSpec((1, 1), lambda i: (0, 0)),
    ],
    out_specs=pl.BlockSpec((NUM_GRAPHS, 1), lambda i: (0, 0)),
    out_shape=jax.ShapeDtypeStruct((NUM_GRAPHS, 1), jnp.float32),
    scratch_shapes=[
        pltpu.VMEM((NUM_GRAPHS, DIM_H), jnp.float32),
        pltpu.VMEM((NUM_GRAPHS, 1), jnp.float32),
    ],
)


NC2 = NC // 2
KCH = 12
SUB = 32


def _sc_w_body(s_hbm, d_hbm, as_hbm, ad_hbm, w_hbm, den2_hbm,
               den_all, as_v, ad_v, sidx_v, didx_v, w_v, den_t,
               den_sum, den_tmp):
    cid = lax.axis_index("c")
    sid = lax.axis_index("s")
    zeros16 = jnp.zeros((16,), jnp.float32)

    pltpu.sync_copy(as_hbm, as_v)
    pltpu.sync_copy(ad_hbm, ad_v)
    pltpu.sync_copy(s_hbm.at[sid, pl.ds(cid * NC2, NC2)], sidx_v)
    pltpu.sync_copy(d_hbm.at[sid, pl.ds(cid * NC2, NC2)], didx_v)

    def _zd(i, _):
        den_t[pl.ds(i * 16, 16)] = zeros16
        return 0
    lax.fori_loop(0, NPAD // 16, _zd, 0)

    def _wchunk(jc, _):
        def _wgrp(g, _):
            sl = pl.ds(g * 16, 16)
            sv = sidx_v[jc, sl]
            dv = didx_v[jc, sl]
            al = plsc.load_gather(as_v, [sv]) + plsc.load_gather(ad_v, [dv])
            al = jnp.where(al > 0.0, al, al * 0.2)
            w = jnp.exp(al)
            w_v[jc, sl] = w
            plsc.addupdate_scatter(den_t, [dv], w)
            return 0
        lax.fori_loop(0, C // 16, _wgrp, 0)
        return 0
    lax.fori_loop(0, NC2, _wchunk, 0)

    pltpu.sync_copy(w_v, w_hbm.at[sid, pl.ds(cid * NC2, NC2)])
    pltpu.sync_copy(den_t, den_all.at[sid])
    plsc.subcore_barrier()

    base = sid * RPT

    def _zs(i, _):
        den_sum[pl.ds(i * 16, 16)] = zeros16
        return 0
    lax.fori_loop(0, RPT // 16, _zs, 0)

    for t in range(NTILE):
        pltpu.sync_copy(den_all.at[t, pl.ds(base, RPT)], den_tmp)

        def _acc(i, _):
            sl = pl.ds(i * 16, 16)
            den_sum[sl] = den_sum[sl] + den_tmp[sl]
            return 0
        lax.fori_loop(0, RPT // 16, _acc, 0)

    pltpu.sync_copy(den_sum, den2_hbm.at[cid, pl.ds(base, RPT)])


_sc_w = pl.kernel(
    _sc_w_body,
    out_type=(
        jax.ShapeDtypeStruct((NTILE, NC, C), jnp.float32),
        jax.ShapeDtypeStruct((2, NPAD), jnp.float32),
    ),
    mesh=plsc.VectorSubcoreMesh(core_axis_name="c", subcore_axis_name="s"),
    compiler_params=pltpu.CompilerParams(needs_layout_passes=False, use_tc_tiling_on_sc=False),
    scratch_types=[
        pltpu.VMEM_SHARED((NTILE, NPAD), jnp.float32),
        pltpu.VMEM((NPAD,), jnp.float32),
        pltpu.VMEM((NPAD,), jnp.float32),
        pltpu.VMEM((NC2, C), jnp.int32),
        pltpu.VMEM((NC2, C), jnp.int32),
        pltpu.VMEM((NC2, C), jnp.float32),
        pltpu.VMEM((NPAD,), jnp.float32),
        pltpu.VMEM((RPT,), jnp.float32),
        pltpu.VMEM((RPT,), jnp.float32),
    ],
)


def _sc_agg_body(s_hbm, d4_hbm, w_hbm, h_hbm, den2_hbm, out_hbm,
                 num_s, sadj_v, didx4_v, w_v, rows0_v, rows1_v, den_sum,
                 den_tmp, gsem0, gsem1, ssem0):
    cid = lax.axis_index("c")
    sid = lax.axis_index("s")
    zeros16 = jnp.zeros((16,), jnp.float32)
    npad_off = cid * NPAD
    rows = (rows0_v, rows1_v)
    gsem = (gsem0, gsem1)
    NSUB = C // SUB

    def _start_gather(j, b):
        pltpu.make_async_copy(h_hbm.at[sadj_v.at[j]], rows[b], gsem[b]).start()

    def _wait_gather(j, b):
        pltpu.make_async_copy(h_hbm.at[sadj_v.at[j]], rows[b], gsem[b]).wait()

    def _scale_block(jj, b, q):
        rv = rows[b]

        def _e(e, _):
            wb = plsc.load_gather(
                w_v, [jnp.full((16,), jj, jnp.int32),
                      jnp.full((16,), q * SUB + e, jnp.int32)])
            for k in range(8):
                sl = pl.ds(k * 16, 16)
                rv[q * SUB + e, sl] = rv[q * SUB + e, sl] * wb
            return 0
        lax.fori_loop(0, SUB, _e, 0)

    def _chunk_body(j, b):
        descs = []
        for q in range(NSUB):
            _scale_block(j, b, q)
            descs.append(pltpu.async_copy(
                rows[b].at[pl.ds(q * SUB, SUB)],
                num_s.at[didx4_v.at[NSUB * j + q]],
                ssem0, add=True))
        for dsc in descs:
            dsc.wait()

    def _zr(r, _):
        for k in range(8):
            rows0_v[r, pl.ds(k * 16, 16)] = zeros16
        return 0
    lax.fori_loop(0, C, _zr, 0)
    for j in range(RPT // C):
        pltpu.sync_copy(rows0_v, num_s.at[pl.ds(sid * RPT + j * C, C)])
    plsc.subcore_barrier()

    def _group(g, _):
        pltpu.sync_copy(s_hbm.at[sid, pl.ds(g * KCH, KCH)], sadj_v)
        pltpu.sync_copy(d4_hbm.at[sid, pl.ds(g * KCH * (C // SUB),
                                             KCH * (C // SUB))], didx4_v)
        pltpu.sync_copy(w_hbm.at[sid, pl.ds(g * KCH, KCH)], w_v)

        def _adj(i, _):
            jj = i // (C // 16)
            sl = pl.ds((i % (C // 16)) * 16, 16)
            sadj_v[jj, sl] = sadj_v[jj, sl] + npad_off
            return 0
        lax.fori_loop(0, KCH * (C // 16), _adj, 0)

        _start_gather(0, 0)

        def _pair(p, _):
            _start_gather(2 * p + 1, 1)
            _wait_gather(2 * p, 0)
            _chunk_body(2 * p, 0)

            @pl.when(p < KCH // 2 - 1)
            def _():
                _start_gather(2 * p + 2, 0)
            _wait_gather(2 * p + 1, 1)
            _chunk_body(2 * p + 1, 1)
            return 0
        lax.fori_loop(0, KCH // 2, _pair, 0)
        return 0
    lax.fori_loop(0, NC // KCH, _group, 0)
    plsc.subcore_barrier()

    base = sid * RPT
    pltpu.sync_copy(den2_hbm.at[0, pl.ds(base, RPT)], den_sum)
    pltpu.sync_copy(den2_hbm.at[1, pl.ds(base, RPT)], den_tmp)

    def _accd(i, _):
        sl = pl.ds(i * 16, 16)
        den_sum[sl] = den_sum[sl] + den_tmp[sl]
        return 0
    lax.fori_loop(0, RPT // 16, _accd, 0)

    for j in range(RPT // C):
        rv = rows[j % 2]
        pltpu.sync_copy(num_s.at[pl.ds(base + j * C, C)], rv)

        def _div(r, _):
            db = plsc.load_gather(
                den_sum, [jnp.full((16,), j * C + r, jnp.int32)]) + 1e-16
            for k in range(8):
                sl = pl.ds(k * 16, 16)
                rv[r, sl] = rv[r, sl] / db
            return 0
        lax.fori_loop(0, C, _div, 0)
        pltpu.sync_copy(rv,
                        out_hbm.at[pl.ds(npad_off + base + j * C, C)])


_sc_agg = pl.kernel(
    _sc_agg_body,
    out_type=jax.ShapeDtypeStruct((2 * NPAD, HALF), jnp.float32),
    mesh=plsc.VectorSubcoreMesh(core_axis_name="c", subcore_axis_name="s"),
    compiler_params=pltpu.CompilerParams(needs_layout_passes=False, use_tc_tiling_on_sc=False),
    scratch_types=[
        pltpu.VMEM_SHARED((NPAD, HALF), jnp.float32),
        pltpu.VMEM((KCH, C), jnp.int32),
        pltpu.VMEM((KCH * (C // SUB), SUB), jnp.int32),
        pltpu.VMEM((KCH, C), jnp.float32),
        pltpu.VMEM((C, HALF), jnp.float32),
        pltpu.VMEM((C, HALF), jnp.float32),
        pltpu.VMEM((RPT,), jnp.float32),
        pltpu.VMEM((RPT,), jnp.float32),
        pltpu.SemaphoreType.DMA,
        pltpu.SemaphoreType.DMA,
        pltpu.SemaphoreType.DMA,
    ],
)


def _sc_edge(s, d, d4, h_flat, as_a, ad_a):
    w, den2 = _sc_w(s, d, as_a, ad_a)
    return _sc_agg(s, d4, w, h_flat, den2)


def kernel(x, edge_index, batch, W1, a_src1, a_dst1, b1, W2, a_src2, a_dst2,
           b2, W3, a_src3, a_dst3, b3, lin_W, lin_b):
    f32 = jnp.float32
    loop = jnp.arange(N, dtype=jnp.int32)
    s = jnp.concatenate([edge_index[0].astype(jnp.int32), loop])
    d = jnp.concatenate([edge_index[1].astype(jnp.int32), loop])
    pad = EP - s.shape[0]
    s = jnp.concatenate([s, jnp.full((pad,), N, jnp.int32)]).reshape(NTILE, NC, C)
    d_flat = jnp.concatenate([d, jnp.full((pad,), N, jnp.int32)])
    d = d_flat.reshape(NTILE, NC, C)
    d4 = d_flat.reshape(NTILE, NC * (C // SUB), SUB)

    x_pad = jnp.zeros((NPAD, DIM_IN), f32).at[:N].set(x)
    batch_p = jnp.full((1, NPAD), NUM_GRAPHS, jnp.int32)
    batch_p = batch_p.at[0, :N].set(batch.astype(jnp.int32))

    av1 = jnp.stack([a_src1, a_dst1])
    av2 = jnp.stack([a_src2, a_dst2])
    av3 = jnp.stack([a_src3, a_dst3])

    h, aa = _prep1(x_pad, W1, av1)
    o = _sc_edge(s, d, d4, h.reshape(2 * NPAD, HALF), aa[0], aa[1])
    h, aa = _prep23(o.reshape(2, NPAD, HALF), b1.reshape(1, DIM_H), W2, av2)
    o = _sc_edge(s, d, d4, h.reshape(2 * NPAD, HALF), aa[0], aa[1])
    h, aa = _prep23(o.reshape(2, NPAD, HALF), b2.reshape(1, DIM_H), W3, av3)
    o = _sc_edge(s, d, d4, h.reshape(2 * NPAD, HALF), aa[0], aa[1])
    return _final(o.reshape(2, NPAD, HALF), b3.reshape(1, DIM_H), batch_p,
                  lin_W.reshape(1, DIM_H), lin_b.reshape(1, 1))


# speedup vs baseline: 1.2707x; 1.0849x over previous
import functools

import jax
import jax.numpy as jnp
from jax import lax
from jax.experimental import pallas as pl
from jax.experimental.pallas import tpu as pltpu
from jax.experimental.pallas import tpu_sc as plsc

N = 10000
DIM_IN = 11
DIM_H = 256
NUM_GRAPHS = 64

NPAD = 10240
R = 1024
GRID = NPAD // R
NTILE = 16
C = 128
NC = 84
T = NC * C
EP = 16 * T
RPT = NPAD // NTILE
HALF = 128


def _prep1_body(x_ref, w_ref, av_ref, h_ref, aa_ref):
    h = jnp.dot(x_ref[...], w_ref[...], preferred_element_type=jnp.float32)
    h_ref[0] = h[:, :HALF]
    h_ref[1] = h[:, HALF:]
    aa_ref[0] = jnp.sum(h * av_ref[0][None, :], axis=-1)
    aa_ref[1] = jnp.sum(h * av_ref[1][None, :], axis=-1)


def _prep23_body(prev_ref, b_ref, w_ref, av_ref, h_ref, aa_ref):
    feat = jnp.concatenate([prev_ref[0], prev_ref[1]], axis=-1) + b_ref[...]
    feat = jnp.maximum(feat, 0.0)
    h = jnp.dot(feat, w_ref[...], preferred_element_type=jnp.float32)
    h_ref[0] = h[:, :HALF]
    h_ref[1] = h[:, HALF:]
    aa_ref[0] = jnp.sum(h * av_ref[0][None, :], axis=-1)
    aa_ref[1] = jnp.sum(h * av_ref[1][None, :], axis=-1)


_prep_out = [
    jax.ShapeDtypeStruct((2, NPAD, HALF), jnp.float32),
    jax.ShapeDtypeStruct((2, NPAD), jnp.float32),
]
_prep_out_specs = [
    pl.BlockSpec((2, R, HALF), lambda i: (0, i, 0)),
    pl.BlockSpec((2, R), lambda i: (0, i)),
]

_prep1 = pl.pallas_call(
    _prep1_body,
    grid=(GRID,),
    in_specs=[
        pl.BlockSpec((R, DIM_IN), lambda i: (i, 0)),
        pl.BlockSpec((DIM_IN, DIM_H), lambda i: (0, 0)),
        pl.BlockSpec((2, DIM_H), lambda i: (0, 0)),
    ],
    out_specs=_prep_out_specs,
    out_shape=_prep_out,
)

_prep23 = pl.pallas_call(
    _prep23_body,
    grid=(GRID,),
    in_specs=[
        pl.BlockSpec((2, R, HALF), lambda i: (0, i, 0)),
        pl.BlockSpec((1, DIM_H), lambda i: (0, 0)),
        pl.BlockSpec((DIM_H, DIM_H), lambda i: (0, 0)),
        pl.BlockSpec((2, DIM_H), lambda i: (0, 0)),
    ],
    out_specs=_prep_out_specs,
    out_shape=_prep_out,
)


def _final_body(prev_ref, b_ref, batch_ref, lw_ref, lb_ref, out_ref,
                pooled_ref, counts_ref):
    i = pl.program_id(0)

    @pl.when(i == 0)
    def _init():
        pooled_ref[...] = jnp.zeros_like(pooled_ref)
        counts_ref[...] = jnp.zeros_like(counts_ref)

    feat = jnp.concatenate([prev_ref[0], prev_ref[1]], axis=-1) + b_ref[...]
    gids = lax.broadcasted_iota(jnp.int32, (NUM_GRAPHS, R), 0)
    onehot = (gids == batch_ref[...]).astype(jnp.float32)
    pooled_ref[...] += jnp.dot(onehot, feat, preferred_element_type=jnp.float32,
                               precision=lax.Precision.HIGHEST)
    counts_ref[...] += jnp.sum(onehot, axis=1, keepdims=True)

    @pl.when(i == GRID - 1)
    def _fin():
        cnt = jnp.maximum(counts_ref[...], 1.0)
        pooled = pooled_ref[...] / cnt
        out_ref[...] = (jnp.dot(pooled, lw_ref[...],
                                preferred_element_type=jnp.float32)
                        + lb_ref[...])


_final = pl.pallas_call(
    _final_body,
    grid=(GRID,),
    in_specs=[
        pl.BlockSpec((2, R, HALF), lambda i: (0, i, 0)),
        pl.BlockSpec((1, DIM_H), lambda i: (0, 0)),
        pl.BlockSpec((1, R), lambda i: (0, i)),
        pl.BlockSpec((DIM_H, 1), lambda i: (0, 0)),
        pl.BlockSpec((1, 1), lambda i: (0, 0)),
    ],
    out_specs=pl.BlockSpec((NUM_GRAPHS, 1), lambda i: (0, 0)),
    out_shape=jax.ShapeDtypeStruct((NUM_GRAPHS, 1), jnp.float32),
    scratch_shapes=[
        pltpu.VMEM((NUM_GRAPHS, DIM_H), jnp.float32),
        pltpu.VMEM((NUM_GRAPHS, 1), jnp.float32),
    ],
)


NC2 = NC // 2
KCH = 12
SUB = 32


def _sc_w_body(s_hbm, d_hbm, as_hbm, ad_hbm, w_hbm, den2_hbm,
               den_all, as_v, ad_v, sidx_v, didx_v, w_v, den_t,
               den_sum, den_tmp):
    cid = lax.axis_index("c")
    sid = lax.axis_index("s")
    zeros16 = jnp.zeros((16,), jnp.float32)

    pltpu.sync_copy(as_hbm, as_v)
    pltpu.sync_copy(ad_hbm, ad_v)
    pltpu.sync_copy(s_hbm.at[sid, pl.ds(cid * NC2, NC2)], sidx_v)
    pltpu.sync_copy(d_hbm.at[sid, pl.ds(cid * NC2, NC2)], didx_v)

    def _zd(i, _):
        den_t[pl.ds(i * 16, 16)] = zeros16
        return 0
    lax.fori_loop(0, NPAD // 16, _zd, 0)

    def _wchunk(jc, _):
        def _wgrp(g, _):
            sl = pl.ds(g * 16, 16)
            sv = sidx_v[jc, sl]
            dv = didx_v[jc, sl]
            al = plsc.load_gather(as_v, [sv]) + plsc.load_gather(ad_v, [dv])
            al = jnp.where(al > 0.0, al, al * 0.2)
            w = jnp.exp(al)
            w_v[jc, sl] = w
            plsc.addupdate_scatter(den_t, [dv], w)
            return 0
        lax.fori_loop(0, C // 16, _wgrp, 0)
        return 0
    lax.fori_loop(0, NC2, _wchunk, 0)

    pltpu.sync_copy(w_v, w_hbm.at[sid, pl.ds(cid * NC2, NC2)])
    pltpu.sync_copy(den_t, den_all.at[sid])
    plsc.subcore_barrier()

    base = sid * RPT

    def _zs(i, _):
        den_sum[pl.ds(i * 16, 16)] = zeros16
        return 0
    lax.fori_loop(0, RPT // 16, _zs, 0)

    for t in range(NTILE):
        pltpu.sync_copy(den_all.at[t, pl.ds(base, RPT)], den_tmp)

        def _acc(i, _):
            sl = pl.ds(i * 16, 16)
            den_sum[sl] = den_sum[sl] + den_tmp[sl]
            return 0
        lax.fori_loop(0, RPT // 16, _acc, 0)

    pltpu.sync_copy(den_sum, den2_hbm.at[cid, pl.ds(base, RPT)])


_sc_w = pl.kernel(
    _sc_w_body,
    out_type=(
        jax.ShapeDtypeStruct((NTILE, NC, C), jnp.float32),
        jax.ShapeDtypeStruct((2, NPAD), jnp.float32),
    ),
    mesh=plsc.VectorSubcoreMesh(core_axis_name="c", subcore_axis_name="s"),
    compiler_params=pltpu.CompilerParams(needs_layout_passes=False, use_tc_tiling_on_sc=False),
    scratch_types=[
        pltpu.VMEM_SHARED((NTILE, NPAD), jnp.float32),
        pltpu.VMEM((NPAD,), jnp.float32),
        pltpu.VMEM((NPAD,), jnp.float32),
        pltpu.VMEM((NC2, C), jnp.int32),
        pltpu.VMEM((NC2, C), jnp.int32),
        pltpu.VMEM((NC2, C), jnp.float32),
        pltpu.VMEM((NPAD,), jnp.float32),
        pltpu.VMEM((RPT,), jnp.float32),
        pltpu.VMEM((RPT,), jnp.float32),
    ],
)


def _sc_agg_body(s_hbm, d4_hbm, w_hbm, h_hbm, den2_hbm, out_hbm,
                 num_s, sadj_v, didx4_v, w_v, rows0_v, rows1_v, den_sum,
                 den_tmp, gsem0, gsem1, ssem0):
    cid = lax.axis_index("c")
    sid = lax.axis_index("s")
    zeros16 = jnp.zeros((16,), jnp.float32)
    npad_off = cid * NPAD
    rows = (rows0_v, rows1_v)
    gsem = (gsem0, gsem1)
    NSUB = C // SUB

    def _start_gather(j, b):
        pltpu.make_async_copy(h_hbm.at[sadj_v.at[j]], rows[b], gsem[b]).start()

    def _wait_gather(j, b):
        pltpu.make_async_copy(h_hbm.at[sadj_v.at[j]], rows[b], gsem[b]).wait()

    def _scale_block(jj, b, q):
        rv = rows[b]

        def _e(e, _):
            wb = plsc.load_gather(
                w_v, [jnp.full((16,), jj, jnp.int32),
                      jnp.full((16,), q * SUB + e, jnp.int32)])
            for k in range(8):
                sl = pl.ds(k * 16, 16)
                rv[q * SUB + e, sl] = rv[q * SUB + e, sl] * wb
            return 0
        lax.fori_loop(0, SUB, _e, 0)

    def _chunk_body(j, b):
        descs = []
        for q in range(NSUB):
            _scale_block(j, b, q)
            descs.append(pltpu.async_copy(
                rows[b].at[pl.ds(q * SUB, SUB)],
                num_s.at[didx4_v.at[NSUB * j + q]],
                ssem0, add=True))
        for dsc in descs:
            dsc.wait()

    def _zr(r, _):
        for k in range(8):
            rows0_v[r, pl.ds(k * 16, 16)] = zeros16
        return 0
    lax.fori_loop(0, C, _zr, 0)
    for j in range(RPT // C):
        pltpu.sync_copy(rows0_v, num_s.at[pl.ds(sid * RPT + j * C, C)])
    plsc.subcore_barrier()

    def _group(g, _):
        pltpu.sync_copy(s_hbm.at[sid, pl.ds(g * KCH, KCH)], sadj_v)
        pltpu.sync_copy(d4_hbm.at[sid, pl.ds(g * KCH * (C // SUB),
                                             KCH * (C // SUB))], didx4_v)
        pltpu.sync_copy(w_hbm.at[sid, pl.ds(g * KCH, KCH)], w_v)

        def _adj(i, _):
            jj = i // (C // 16)
            sl = pl.ds((i % (C // 16)) * 16, 16)
            sadj_v[jj, sl] = sadj_v[jj, sl] + npad_off
            return 0
        lax.fori_loop(0, KCH * (C // 16), _adj, 0)

        _start_gather(0, 0)

        def _pair(p, _):
            _start_gather(2 * p + 1, 1)
            _wait_gather(2 * p, 0)
            _chunk_body(2 * p, 0)

            @pl.when(p < KCH // 2 - 1)
            def _():
                _start_gather(2 * p + 2, 0)
            _wait_gather(2 * p + 1, 1)
            _chunk_body(2 * p + 1, 1)
            return 0
        lax.fori_loop(0, KCH // 2, _pair, 0)
        return 0
    lax.fori_loop(0, NC // KCH, _group, 0)
    plsc.subcore_barrier()

    base = sid * RPT
    pltpu.sync_copy(den2_hbm.at[0, pl.ds(base, RPT)], den_sum)
    pltpu.sync_copy(den2_hbm.at[1, pl.ds(base, RPT)], den_tmp)

    def _accd(i, _):
        sl = pl.ds(i * 16, 16)
        den_sum[sl] = den_sum[sl] + den_tmp[sl]
        return 0
    lax.fori_loop(0, RPT // 16, _accd, 0)

    for j in range(RPT // C):
        rv = rows[j % 2]
        pltpu.sync_copy(num_s.at[pl.ds(base + j * C, C)], rv)

        def _div(r, _):
            db = plsc.load_gather(
                den_sum, [jnp.full((16,), j * C + r, jnp.int32)]) + 1e-16
            for k in range(8):
                sl = pl.ds(k * 16, 16)
                rv[r, sl] = rv[r, sl] / db
            return 0
        lax.fori_loop(0, C, _div, 0)
        pltpu.sync_copy(rv,
                        out_hbm.at[pl.ds(npad_off + base + j * C, C)])


_sc_agg = pl.kernel(
    _sc_agg_body,
    out_type=jax.ShapeDtypeStruct((2 * NPAD, HALF), jnp.float32),
    mesh=plsc.VectorSubcoreMesh(core_axis_name="c", subcore_axis_name="s"),
    compiler_params=pltpu.CompilerParams(needs_layout_passes=False, use_tc_tiling_on_sc=False),
    scratch_types=[
        pltpu.VMEM_SHARED((NPAD, HALF), jnp.float32),
        pltpu.VMEM((KCH, C), jnp.int32),
        pltpu.VMEM((KCH * (C // SUB), SUB), jnp.int32),
        pltpu.VMEM((KCH, C), jnp.float32),
        pltpu.VMEM((C, HALF), jnp.float32),
        pltpu.VMEM((C, HALF), jnp.float32),
        pltpu.VMEM((RPT,), jnp.float32),
        pltpu.VMEM((RPT,), jnp.float32),
        pltpu.SemaphoreType.DMA,
        pltpu.SemaphoreType.DMA,
        pltpu.SemaphoreType.DMA,
    ],
)


def _sc_edge(s, d, d4, h_flat, as_a, ad_a):
    w, den2 = _sc_w(s, d, as_a, ad_a)
    return _sc_agg(s, d4, w, h_flat, den2)


def kernel(x, edge_index, batch, W1, a_src1, a_dst1, b1, W2, a_src2, a_dst2,
           b2, W3, a_src3, a_dst3, b3, lin_W, lin_b):
    f32 = jnp.float32
    loop = jnp.arange(N, dtype=jnp.int32)
    s = jnp.concatenate([edge_index[0].astype(jnp.int32), loop])
    d = jnp.concatenate([edge_index[1].astype(jnp.int32), loop])
    pad = EP - s.shape[0]
    s = jnp.concatenate([s, jnp.full((pad,), N, jnp.int32)]).reshape(NTILE, NC, C)
    d_flat = jnp.concatenate([d, jnp.full((pad,), N, jnp.int32)])
    d = d_flat.reshape(NTILE, NC, C)
    d4 = d_flat.reshape(NTILE, NC * (C // SUB), SUB)

    x_pad = jnp.zeros((NPAD, DIM_IN), f32).at[:N].set(x)
    batch_p = jnp.full((1, NPAD), NUM_GRAPHS, jnp.int32)
    batch_p = batch_p.at[0, :N].set(batch.astype(jnp.int32))

    av1 = jnp.stack([a_src1, a_dst1])
    av2 = jnp.stack([a_src2, a_dst2])
    av3 = jnp.stack([a_src3, a_dst3])

    h, aa = _prep1(x_pad, W1, av1)
    o = _sc_edge(s, d, d4, h.reshape(2 * NPAD, HALF), aa[0], aa[1])
    h, aa = _prep23(o.reshape(2, NPAD, HALF), b1.reshape(1, DIM_H), W2, av2)
    o = _sc_edge(s, d, d4, h.reshape(2 * NPAD, HALF), aa[0], aa[1])
    h, aa = _prep23(o.reshape(2, NPAD, HALF), b2.reshape(1, DIM_H), W3, av3)
    o = _sc_edge(s, d, d4, h.reshape(2 * NPAD, HALF), aa[0], aa[1])
    return _final(o.reshape(2, NPAD, HALF), b3.reshape(1, DIM_H), batch_p,
                  lin_W, lin_b.reshape(1, 1))
